# Initial kernel scaffold; baseline (speedup 1.0000x reference)
import jax
import jax.numpy as jnp
from jax.experimental import pallas as pl


def kernel(z, edge_index, edge_attr, edge_length, emb_W0, emb_b0, emb_W1, emb_b1, m1_W0, m1_b0, m1_W1, m1_b1, m2_W0, m2_b0, m2_W1, m2_b1):
    raise NotImplementedError("write your pallas kernel here")


# speedup vs baseline: 2.5539x; 2.5539x over previous
import functools

import jax
import jax.numpy as jnp
from jax import lax
from jax.experimental import pallas as pl
from jax.experimental.pallas import tpu as pltpu
from jax.experimental.pallas import tpu_sc as plsc

N = 10000
E = 320000
H = 128
CUTOFF = 10.0
NUM_CONVS = 3

NCORE = 2
NSUB = 16
NW = NCORE * NSUB
EW = E // NW
K = 80
NCH = EW // K
N_PAD = 10240
ROWS_PER_TILE = N_PAD // NSUB
LANES = 16
VPR = H // LANES


def _mlp_body(x, w0, b0, w1, b1):
    h = jnp.maximum(jnp.dot(x, w0, preferred_element_type=jnp.float32) + b0, 0.0)
    return jnp.dot(h, w1, preferred_element_type=jnp.float32) + b1


def _emb_kernel(z_ref, w0_ref, b0_ref, w1_ref, b1_ref, o_ref):
    o_ref[...] = _mlp_body(z_ref[...], w0_ref[...], b0_ref[...],
                           w1_ref[...], b1_ref[...])


def _edge_kernel(ea_ref, el_ref, w0_ref, b0_ref, w1_ref, b1_ref, o_ref):
    y = _mlp_body(ea_ref[...], w0_ref[...], b0_ref[...], w1_ref[...], b1_ref[...])
    o_ref[...] = jnp.where(el_ref[...] <= CUTOFF, y, 0.0)


def _update_kernel(parts_ref0, parts_ref1, cv_ref, w0_ref, b0_ref, w1_ref,
                   b1_ref, o_ref, *, apply_relu):
    cv = cv_ref[...]
    out = parts_ref0[0] + parts_ref1[0] + cv
    y = _mlp_body(out, w0_ref[...], b0_ref[...], w1_ref[...], b1_ref[...])
    if apply_relu:
        y = jnp.maximum(y, 0.0)
    o_ref[...] = y + cv


def _full_spec(shape):
    return pl.BlockSpec(shape, lambda i: (0,) * len(shape))


def _emb(z, w0, b0, w1, b1):
    bn = 2000
    return pl.pallas_call(
        _emb_kernel,
        grid=(N // bn,),
        in_specs=[
            pl.BlockSpec((bn, z.shape[1]), lambda i: (i, 0)),
            _full_spec(w0.shape), _full_spec(b0.shape),
            _full_spec(w1.shape), _full_spec(b1.shape),
        ],
        out_specs=pl.BlockSpec((bn, H), lambda i: (i, 0)),
        out_shape=jax.ShapeDtypeStruct((N, H), jnp.float32),
    )(z, w0, b0, w1, b1)


def _edge_mlp(ea, el, w0, b0, w1, b1):
    be = 4000
    return pl.pallas_call(
        _edge_kernel,
        grid=(E // be,),
        in_specs=[
            pl.BlockSpec((be, H), lambda i: (i, 0)),
            pl.BlockSpec((be, 1), lambda i: (i, 0)),
            _full_spec(w0.shape), _full_spec(b0.shape),
            _full_spec(w1.shape), _full_spec(b1.shape),
        ],
        out_specs=pl.BlockSpec((be, H), lambda i: (i, 0)),
        out_shape=jax.ShapeDtypeStruct((E, H), jnp.float32),
    )(ea, el, w0, b0, w1, b1)


def _update(parts, cv, w0, b0, w1, b1, apply_relu):
    bn = 2000
    return pl.pallas_call(
        functools.partial(_update_kernel, apply_relu=apply_relu),
        grid=(N // bn,),
        in_specs=[
            pl.BlockSpec((1, bn, H), lambda i: (0, i, 0)),
            pl.BlockSpec((1, bn, H), lambda i: (1, i, 0)),
            pl.BlockSpec((bn, H), lambda i: (i, 0)),
            _full_spec(w0.shape), _full_spec(b0.shape),
            _full_spec(w1.shape), _full_spec(b1.shape),
        ],
        out_specs=pl.BlockSpec((bn, H), lambda i: (i, 0)),
        out_shape=jax.ShapeDtypeStruct((N, H), jnp.float32),
    )(parts, parts, cv, w0, b0, w1, b1)


@functools.cache
def _make_sc_propagate():
    mesh = plsc.VectorSubcoreMesh(core_axis_name="c", subcore_axis_name="s",
                                  num_cores=NCORE, num_subcores=NSUB)
    return pl.kernel(
        _sc_propagate_body,
        out_type=jax.ShapeDtypeStruct((NCORE, N_PAD, H), jnp.float32),
        mesh=mesh,
        scratch_types=[
            pltpu.VMEM((1, K), jnp.int32),
            pltpu.VMEM((1, K), jnp.int32),
            pltpu.VMEM((K, H), jnp.float32),
            pltpu.VMEM((K, H), jnp.float32),
            pltpu.VMEM_SHARED((N_PAD, H), jnp.float32),
            pltpu.SemaphoreType.DMA,
        ],
    )


def _sc_propagate_body(x_hbm, wm_hbm, src_hbm, dst_hbm, out_hbm,
                       src_v, dst_v, xbuf, wbuf, acc, sem):
    cid = lax.axis_index("c")
    sid = lax.axis_index("s")
    wid = cid * NSUB + sid

    def _zero_row(e, _):
        for kk in range(VPR):
            wbuf[e, pl.ds(kk * LANES, LANES)] = jnp.zeros((LANES,), jnp.float32)
        return 0
    lax.fori_loop(0, K, _zero_row, 0)
    for r in range(ROWS_PER_TILE // K):
        pltpu.sync_copy(wbuf, acc.at[pl.ds(sid * ROWS_PER_TILE + r * K, K)])

    plsc.subcore_barrier()

    def _chunk(j, _):
        pltpu.sync_copy(src_hbm.at[wid, j], src_v)
        pltpu.sync_copy(dst_hbm.at[wid, j], dst_v)
        pltpu.async_copy(x_hbm.at[src_v.at[0]], xbuf, sem).wait()
        pltpu.sync_copy(wm_hbm.at[wid, j], wbuf)

        def _row(e, _):
            for kk in range(VPR):
                sl = pl.ds(kk * LANES, LANES)
                wbuf[e, sl] = jnp.maximum(xbuf[e, sl] + wbuf[e, sl], 0.0)
            return 0
        lax.fori_loop(0, K, _row, 0)

        pltpu.sync_copy(wbuf, acc.at[dst_v.at[0]], add=True)
        return 0

    lax.fori_loop(0, NCH, _chunk, 0)
    plsc.subcore_barrier()

    sl = pl.ds(sid * ROWS_PER_TILE, ROWS_PER_TILE)
    pltpu.sync_copy(acc.at[sl], out_hbm.at[cid, sl])


def kernel(z, edge_index, edge_attr, edge_length,
           emb_W0, emb_b0, emb_W1, emb_b1,
           m1_W0, m1_b0, m1_W1, m1_b1,
           m2_W0, m2_b0, m2_W1, m2_b1):
    x = _emb(z, emb_W0, emb_b0.reshape(1, H), emb_W1, emb_b1.reshape(1, H))
    wm = _edge_mlp(edge_attr, edge_length.reshape(E, 1),
                   m2_W0, m2_b0.reshape(1, H), m2_W1, m2_b1.reshape(1, H))
    wm4 = wm.reshape(NW, NCH, K, H)
    src = edge_index[0].astype(jnp.int32).reshape(NW, NCH, 1, K)
    dst = edge_index[1].astype(jnp.int32).reshape(NW, NCH, 1, K)

    conv = x
    for conv_id in range(NUM_CONVS):
        parts = _make_sc_propagate()(conv, wm4, src, dst)
        conv = _update(parts, conv, m1_W0, m1_b0.reshape(1, H),
                       m1_W1, m1_b1.reshape(1, H),
                       apply_relu=conv_id < NUM_CONVS - 1)
    return conv


# speedup vs baseline: 4.7828x; 1.8727x over previous
import functools

import jax
import jax.numpy as jnp
from jax import lax
from jax.experimental import pallas as pl
from jax.experimental.pallas import tpu as pltpu
from jax.experimental.pallas import tpu_sc as plsc

N = 10000
E = 320000
H = 128
CUTOFF = 10.0
NUM_CONVS = 3

NCORE = 2
NSUB = 16
NW = NCORE * NSUB
EW = E // NW
K = 80
NCH = EW // K
N_PAD = 10240
ROWS_PER_TILE = N_PAD // NSUB
LANES = 16
VPR = H // LANES


def _mlp_body(x, w0, b0, w1, b1):
    h = jnp.maximum(jnp.dot(x, w0, preferred_element_type=jnp.float32) + b0, 0.0)
    return jnp.dot(h, w1, preferred_element_type=jnp.float32) + b1


def _emb_kernel(z_ref, w0_ref, b0_ref, w1_ref, b1_ref, o_ref):
    o_ref[...] = _mlp_body(z_ref[...], w0_ref[...], b0_ref[...],
                           w1_ref[...], b1_ref[...])


def _edge_kernel(ea_ref, el_ref, w0_ref, b0_ref, w1_ref, b1_ref, o_ref):
    y = _mlp_body(ea_ref[...], w0_ref[...], b0_ref[...], w1_ref[...], b1_ref[...])
    o_ref[...] = jnp.where(el_ref[...] <= CUTOFF, y, 0.0)


def _update_kernel(parts_ref0, parts_ref1, cv_ref, w0_ref, b0_ref, w1_ref,
                   b1_ref, o_ref, *, apply_relu):
    cv = cv_ref[...]
    out = parts_ref0[0] + parts_ref1[0] + cv
    y = _mlp_body(out, w0_ref[...], b0_ref[...], w1_ref[...], b1_ref[...])
    if apply_relu:
        y = jnp.maximum(y, 0.0)
    o_ref[...] = y + cv


def _full_spec(shape):
    return pl.BlockSpec(shape, lambda i: (0,) * len(shape))


def _emb(z, w0, b0, w1, b1):
    bn = 2000
    return pl.pallas_call(
        _emb_kernel,
        grid=(N // bn,),
        in_specs=[
            pl.BlockSpec((bn, z.shape[1]), lambda i: (i, 0)),
            _full_spec(w0.shape), _full_spec(b0.shape),
            _full_spec(w1.shape), _full_spec(b1.shape),
        ],
        out_specs=pl.BlockSpec((bn, H), lambda i: (i, 0)),
        out_shape=jax.ShapeDtypeStruct((N, H), jnp.float32),
    )(z, w0, b0, w1, b1)


def _edge_mlp(ea, el, w0, b0, w1, b1):
    be = 4000
    return pl.pallas_call(
        _edge_kernel,
        grid=(E // be,),
        in_specs=[
            pl.BlockSpec((be, H), lambda i: (i, 0)),
            pl.BlockSpec((be, 1), lambda i: (i, 0)),
            _full_spec(w0.shape), _full_spec(b0.shape),
            _full_spec(w1.shape), _full_spec(b1.shape),
        ],
        out_specs=pl.BlockSpec((be, H), lambda i: (i, 0)),
        out_shape=jax.ShapeDtypeStruct((E, H), jnp.float32),
    )(ea, el, w0, b0, w1, b1)


def _update(parts, cv, w0, b0, w1, b1, apply_relu):
    bn = 2000
    return pl.pallas_call(
        functools.partial(_update_kernel, apply_relu=apply_relu),
        grid=(N // bn,),
        in_specs=[
            pl.BlockSpec((1, bn, H), lambda i: (0, i, 0)),
            pl.BlockSpec((1, bn, H), lambda i: (1, i, 0)),
            pl.BlockSpec((bn, H), lambda i: (i, 0)),
            _full_spec(w0.shape), _full_spec(b0.shape),
            _full_spec(w1.shape), _full_spec(b1.shape),
        ],
        out_specs=pl.BlockSpec((bn, H), lambda i: (i, 0)),
        out_shape=jax.ShapeDtypeStruct((N, H), jnp.float32),
    )(parts, parts, cv, w0, b0, w1, b1)


@functools.cache
def _make_sc_propagate():
    mesh = plsc.VectorSubcoreMesh(core_axis_name="c", subcore_axis_name="s",
                                  num_cores=NCORE, num_subcores=NSUB)
    return pl.kernel(
        _sc_propagate_body,
        out_type=jax.ShapeDtypeStruct((NCORE, N_PAD, H), jnp.float32),
        mesh=mesh,
        scratch_types=[
            pltpu.VMEM((1, K), jnp.int32),
            pltpu.VMEM((1, K), jnp.int32),
            pltpu.VMEM((1, K), jnp.int32),
            pltpu.VMEM((1, K), jnp.int32),
            pltpu.VMEM((K, H), jnp.float32),
            pltpu.VMEM((K, H), jnp.float32),
            pltpu.VMEM((K, H), jnp.float32),
            pltpu.VMEM((K, H), jnp.float32),
            pltpu.VMEM_SHARED((N_PAD, H), jnp.float32),
            pltpu.SemaphoreType.DMA,
            pltpu.SemaphoreType.DMA,
            pltpu.SemaphoreType.DMA,
            pltpu.SemaphoreType.DMA,
        ],
    )


def _sc_propagate_body(x_hbm, wm_hbm, src_hbm, dst_hbm, out_hbm,
                       sv0, sv1, dv0, dv1, xb0, xb1, wb0, wb1,
                       acc, si0, si1, sd0, sd1):
    cid = lax.axis_index("c")
    sid = lax.axis_index("s")
    wid = cid * NSUB + sid
    SV, DV, XB, WB, SI, SD = (sv0, sv1), (dv0, dv1), (xb0, xb1), (wb0, wb1), \
        (si0, si1), (sd0, sd1)

    @plsc.parallel_loop(0, K)
    def _zero_row(e):
        for kk in range(VPR):
            wb0[e, pl.ds(kk * LANES, LANES)] = jnp.zeros((LANES,), jnp.float32)
    for r in range(ROWS_PER_TILE // K):
        pltpu.sync_copy(wb0, acc.at[pl.ds(sid * ROWS_PER_TILE + r * K, K)])

    plsc.subcore_barrier()

    def _issue_idx(j, b):
        pltpu.async_copy(src_hbm.at[wid, j], SV[b], SI[b])
        pltpu.async_copy(dst_hbm.at[wid, j], DV[b], SI[b])

    def _wait_idx(b):
        pltpu.make_async_copy(src_hbm.at[wid, 0], SV[b], SI[b]).wait()
        pltpu.make_async_copy(dst_hbm.at[wid, 0], DV[b], SI[b]).wait()

    def _issue_data(j, b):
        pltpu.async_copy(x_hbm.at[SV[b].at[0]], XB[b], SD[b])
        pltpu.async_copy(wm_hbm.at[wid, j], WB[b], SD[b])

    def _wait_data(b):
        pltpu.make_async_copy(wm_hbm.at[wid, 0], XB[b], SD[b]).wait()
        pltpu.make_async_copy(wm_hbm.at[wid, 0], WB[b], SD[b]).wait()

    def _compute(b):
        xbuf, wbuf = XB[b], WB[b]

        @plsc.parallel_loop(0, K, unroll=2)
        def _row(e):
            for kk in range(VPR):
                sl = pl.ds(kk * LANES, LANES)
                wbuf[e, sl] = jnp.maximum(xbuf[e, sl] + wbuf[e, sl], 0.0)

    def _scatter(b):
        pltpu.sync_copy(WB[b], acc.at[DV[b].at[0]], add=True)

    _issue_idx(0, 0)
    _wait_idx(0)
    _issue_data(0, 0)
    _issue_idx(1, 1)

    def _steady(j2, _):
        for b in range(2):
            j = j2 * 2 + b
            bn = 1 - b
            _wait_data(b)
            _wait_idx(bn)
            _issue_data(j + 1, bn)
            _compute(b)
            _scatter(b)
            if b == 0:
                _issue_idx(j + 2, b)
            else:
                @pl.when(j + 2 < NCH)
                def _():
                    _issue_idx(j + 2, b)
        return 0

    lax.fori_loop(0, (NCH - 1) // 2, _steady, 0)
    _wait_data(0)
    _compute(0)
    _scatter(0)
    plsc.subcore_barrier()

    sl = pl.ds(sid * ROWS_PER_TILE, ROWS_PER_TILE)
    pltpu.sync_copy(acc.at[sl], out_hbm.at[cid, sl])


def kernel(z, edge_index, edge_attr, edge_length,
           emb_W0, emb_b0, emb_W1, emb_b1,
           m1_W0, m1_b0, m1_W1, m1_b1,
           m2_W0, m2_b0, m2_W1, m2_b1):
    x = _emb(z, emb_W0, emb_b0.reshape(1, H), emb_W1, emb_b1.reshape(1, H))
    wm = _edge_mlp(edge_attr, edge_length.reshape(E, 1),
                   m2_W0, m2_b0.reshape(1, H), m2_W1, m2_b1.reshape(1, H))
    wm4 = wm.reshape(NW, NCH, K, H)
    src = edge_index[0].astype(jnp.int32).reshape(NW, NCH, 1, K)
    dst = edge_index[1].astype(jnp.int32).reshape(NW, NCH, 1, K)

    conv = x
    for conv_id in range(NUM_CONVS):
        parts = _make_sc_propagate()(conv, wm4, src, dst)
        conv = _update(parts, conv, m1_W0, m1_b0.reshape(1, H),
                       m1_W1, m1_b1.reshape(1, H),
                       apply_relu=conv_id < NUM_CONVS - 1)
    return conv
